# Initial kernel scaffold; baseline (speedup 1.0000x reference)
import jax
import jax.numpy as jnp
from jax.experimental import pallas as pl


def kernel(x, edge_index, params):
    raise NotImplementedError("write your pallas kernel here")


# speedup vs baseline: 4.3170x; 4.3170x over previous
import jax
import jax.numpy as jnp
from jax import lax
from jax.experimental import pallas as pl
from jax.experimental.pallas import tpu as pltpu
from jax.experimental.pallas import tpu_sc as plsc

NC, NS = 2, 16
CHUNK = 32
GROUP = 128
RG = 4
PAD_ROWS = 64
EPS = 1e-5
BN = 1000


def _mesh():
    return plsc.VectorSubcoreMesh(core_axis_name="c", subcore_axis_name="s",
                                  num_cores=NC, num_subcores=NS)


def _make_agg(n_chunks, n_acc, n_out, e_pad):
    per_core = n_chunks // NC
    g_total = e_pad // GROUP
    g_tile = g_total // NS
    nwin = g_tile // RG
    z_sl = n_out // NS

    def chunk_pass(tbl, out, src2, dst2, zeros, acc, sbuf, dbuf, rbuf, sem, s):
        pltpu.sync_copy(zeros, acc.at[pl.ds(s * z_sl, z_sl)])
        plsc.subcore_barrier()
        g0 = s * g_tile

        def win(w, carry):
            g = g0 + w * RG
            pltpu.sync_copy(src2.at[pl.ds(g, RG)], sbuf)
            pltpu.sync_copy(dst2.at[pl.ds(g, RG)], dbuf)
            descs = [
                pltpu.async_copy(tbl.at[sbuf.at[r]],
                                 rbuf.at[pl.ds(r * GROUP, GROUP)], sem)
                for r in range(RG)
            ]
            for d in descs:
                d.wait()
            for r in range(RG):
                pltpu.sync_copy(rbuf.at[pl.ds(r * GROUP, GROUP)],
                                acc.at[dbuf.at[r]], add=True)
            return carry

        lax.fori_loop(0, nwin, win, 0)
        plsc.subcore_barrier()
        pltpu.sync_copy(acc.at[pl.ds(s * z_sl, z_sl)],
                        out.at[pl.ds(s * z_sl, z_sl)])
        plsc.subcore_barrier()

    def body(*refs):
        tbls = refs[:n_chunks]
        src2, dst2, zeros = refs[n_chunks:n_chunks + 3]
        outs = refs[n_chunks + 3:2 * n_chunks + 3]
        acc, sbuf, dbuf, rbuf, sem = refs[2 * n_chunks + 3:]
        c = lax.axis_index("c")
        s = lax.axis_index("s")
        for cc in range(NC):
            def run(cc=cc):
                for p in range(per_core):
                    j = cc * per_core + p
                    chunk_pass(tbls[j], outs[j], src2, dst2, zeros,
                               acc, sbuf, dbuf, rbuf, sem, s)
            pl.when(c == cc)(run)

    out_type = tuple(jax.ShapeDtypeStruct((n_out, CHUNK), jnp.float32)
                     for _ in range(n_chunks))
    scratch = [
        pltpu.VMEM_SHARED((n_acc, CHUNK), jnp.float32),
        pltpu.VMEM((RG, GROUP), jnp.int32),
        pltpu.VMEM((RG, GROUP), jnp.int32),
        pltpu.VMEM((RG * GROUP, CHUNK), jnp.float32),
        pltpu.SemaphoreType.DMA,
    ]
    return pl.kernel(body, out_type=out_type, mesh=_mesh(),
                     scratch_types=scratch,
                     compiler_params=pltpu.CompilerParams(
                         use_tc_tiling_on_sc=False))


def _make_cnt(n_acc, n_out, e_pad):
    g_total = e_pad // GROUP
    g_core = g_total // NC
    g_tile = g_core // NS
    z_sl = n_out // NS

    def body(dst2, zeros1, out0, out1, acc, dbuf, ones, sem):
        c = lax.axis_index("c")
        s = lax.axis_index("s")
        for k in range(GROUP // 16):
            ones[pl.ds(k * 16, 16)] = jnp.ones((16,), jnp.float32)
        pltpu.sync_copy(zeros1, acc.at[pl.ds(s * z_sl, z_sl)])
        plsc.subcore_barrier()

        def win(w, carry):
            g = c * g_core + s * g_tile + w
            pltpu.sync_copy(dst2.at[pl.ds(g, 1)], dbuf)
            pltpu.sync_copy(ones, acc.at[dbuf.at[0]], add=True)
            return carry

        lax.fori_loop(0, g_tile, win, 0)
        plsc.subcore_barrier()
        for cc, o in enumerate((out0, out1)):
            def run(cc=cc, o=o):
                pltpu.sync_copy(acc.at[pl.ds(s * z_sl, z_sl)],
                                o.at[pl.ds(s * z_sl, z_sl)])
            pl.when(c == cc)(run)

    out_type = tuple(jax.ShapeDtypeStruct((n_out,), jnp.float32)
                     for _ in range(NC))
    scratch = [
        pltpu.VMEM_SHARED((n_acc,), jnp.float32),
        pltpu.VMEM((1, GROUP), jnp.int32),
        pltpu.VMEM((GROUP,), jnp.float32),
        pltpu.SemaphoreType.DMA,
    ]
    return pl.kernel(body, out_type=out_type, mesh=_mesh(),
                     scratch_types=scratch,
                     compiler_params=pltpu.CompilerParams(
                         use_tc_tiling_on_sc=False))


def _mm_call(aggs, hcs, cnt, wl, wr, bl, n):
    nb = n // BN
    na, nh = len(aggs), len(hcs)

    def kern(*refs):
        agg_r = refs[:na]
        h_r = refs[na:na + nh]
        cnt_r, wl_r, wr_r, bl_r, z_r, st_r = refs[na + nh:]
        a = jnp.concatenate([r[...] for r in agg_r], axis=1)
        h = jnp.concatenate([r[...] for r in h_r], axis=1)
        cb = cnt_r[...]
        inv = 1.0 / jnp.maximum(cb[:, 0] + cb[:, 1], 1.0)
        z = (jnp.dot(a, wl_r[...], preferred_element_type=jnp.float32)
             * inv[:, None]
             + jnp.dot(h, wr_r[...], preferred_element_type=jnp.float32)
             + bl_r[...])
        z_r[...] = z
        st_r[...] = jnp.stack([jnp.sum(z, axis=0),
                               jnp.sum(z * z, axis=0)])[None]

    d = wl.shape[0]
    in_specs = (
        [pl.BlockSpec((BN, CHUNK), lambda i: (i, 0)) for _ in range(na)]
        + [pl.BlockSpec((BN, CHUNK), lambda i: (i, 0)) for _ in range(nh)]
        + [pl.BlockSpec((BN, NC), lambda i: (i, 0)),
           pl.BlockSpec((d, 128), lambda i: (0, 0)),
           pl.BlockSpec((d, 128), lambda i: (0, 0)),
           pl.BlockSpec((1, 128), lambda i: (0, 0))]
    )
    z, st = pl.pallas_call(
        kern,
        grid=(nb,),
        in_specs=in_specs,
        out_specs=(pl.BlockSpec((BN, 128), lambda i: (i, 0)),
                   pl.BlockSpec((1, 2, 128), lambda i: (i, 0, 0))),
        out_shape=(jax.ShapeDtypeStruct((n, 128), jnp.float32),
                   jax.ShapeDtypeStruct((nb, 2, 128), jnp.float32)),
    )(*aggs, *hcs, cnt, wl, wr, bl.reshape(1, 128))
    return z, st


def _bn_call(z, st, g, b, n):
    nb = n // BN

    def kern(z_r, st_r, g_r, b_r, *outs):
        stats = st_r[...]
        mu = jnp.sum(stats[:, 0, :], axis=0) * (1.0 / n)
        ex2 = jnp.sum(stats[:, 1, :], axis=0) * (1.0 / n)
        var = ex2 - mu * mu
        h = jnp.maximum(
            g_r[...] * (z_r[...] - mu[None, :])
            / jnp.sqrt(var + EPS)[None, :] + b_r[...], 0.0)
        for j, o in enumerate(outs):
            o[...] = h[:, j * CHUNK:(j + 1) * CHUNK]

    n_ch = 128 // CHUNK
    outs = pl.pallas_call(
        kern,
        grid=(nb,),
        in_specs=[pl.BlockSpec((BN, 128), lambda i: (i, 0)),
                  pl.BlockSpec((nb, 2, 128), lambda i: (0, 0, 0)),
                  pl.BlockSpec((1, 128), lambda i: (0, 0)),
                  pl.BlockSpec((1, 128), lambda i: (0, 0))],
        out_specs=tuple(pl.BlockSpec((BN, CHUNK), lambda i: (i, 0))
                        for _ in range(n_ch)),
        out_shape=tuple(jax.ShapeDtypeStruct((n, CHUNK), jnp.float32)
                        for _ in range(n_ch)),
    )(z, st, g.reshape(1, 128), b.reshape(1, 128))
    return list(outs)


def _bn_head_call(z, st, g, b, w1, b1, w2, b2, n):
    nb = n // BN

    def kern(z_r, st_r, g_r, b_r, w1_r, b1_r, w2_r, b2_r, o_r):
        stats = st_r[...]
        mu = jnp.sum(stats[:, 0, :], axis=0) * (1.0 / n)
        ex2 = jnp.sum(stats[:, 1, :], axis=0) * (1.0 / n)
        var = ex2 - mu * mu
        h = jnp.maximum(
            g_r[...] * (z_r[...] - mu[None, :])
            / jnp.sqrt(var + EPS)[None, :] + b_r[...], 0.0)
        h1 = jnp.maximum(
            jnp.dot(h, w1_r[...], preferred_element_type=jnp.float32)
            + b1_r[...], 0.0)
        o = jnp.sum(h1 * w2_r[...], axis=1, keepdims=True) + b2_r[...]
        o_r[...] = jax.nn.sigmoid(o)

    hd = w1.shape[1]
    out = pl.pallas_call(
        kern,
        grid=(nb,),
        in_specs=[pl.BlockSpec((BN, 128), lambda i: (i, 0)),
                  pl.BlockSpec((nb, 2, 128), lambda i: (0, 0, 0)),
                  pl.BlockSpec((1, 128), lambda i: (0, 0)),
                  pl.BlockSpec((1, 128), lambda i: (0, 0)),
                  pl.BlockSpec((128, hd), lambda i: (0, 0)),
                  pl.BlockSpec((1, hd), lambda i: (0, 0)),
                  pl.BlockSpec((1, hd), lambda i: (0, 0)),
                  pl.BlockSpec((1, 1), lambda i: (0, 0))],
        out_specs=pl.BlockSpec((BN, 1), lambda i: (i, 0)),
        out_shape=jax.ShapeDtypeStruct((n, 1), jnp.float32),
    )(z, st, g.reshape(1, 128), b.reshape(1, 128), w1,
      b1.reshape(1, hd), w2.reshape(1, hd), b2.reshape(1, 1))
    return out


def kernel(x, edge_index, params):
    n, in_dim = x.shape
    e = edge_index.shape[1]
    src, dst = edge_index[0], edge_index[1]

    unit = GROUP * NS * RG * NC
    e_pad = ((e + unit - 1) // unit) * unit
    pad = e_pad - e
    z_sl = ((-(-n // NS) + 127) // 128) * 128
    n_out = z_sl * NS
    n_acc = max(n_out, n + PAD_ROWS)

    ar = jnp.arange(pad, dtype=jnp.int32)
    src2 = jnp.concatenate([src, ar % n]).reshape(-1, GROUP)
    dst2 = jnp.concatenate([dst, n + (ar % PAD_ROWS)]).reshape(-1, GROUP)
    zeros32 = jnp.zeros((z_sl, CHUNK), jnp.float32)
    zeros1 = jnp.zeros((z_sl,), jnp.float32)

    c0, c1 = _make_cnt(n_acc, n_out, e_pad)(dst2, zeros1)
    cnt = jnp.stack([c0, c1], axis=1)

    agg2 = _make_agg(2, n_acc, n_out, e_pad)
    agg4 = _make_agg(4, n_acc, n_out, e_pad)

    hc = [lax.slice(x, (0, j * CHUNK), (n, (j + 1) * CHUNK))
          for j in range(in_dim // CHUNK)]
    for i in range(3):
        aggs = (agg2 if len(hc) == 2 else agg4)(*hc, src2, dst2, zeros32)
        z, st = _mm_call(list(aggs), hc, cnt, params['Wl%d' % i],
                         params['Wr%d' % i], params['bl%d' % i], n)
        if i < 2:
            hc = _bn_call(z, st, params['bn_g%d' % i],
                          params['bn_b%d' % i], n)
        else:
            out = _bn_head_call(z, st, params['bn_g%d' % i],
                                params['bn_b%d' % i], params['fc1_W'],
                                params['fc1_b'], params['fc2_W'],
                                params['fc2_b'], n)
    return out


# speedup vs baseline: 4.8033x; 1.1126x over previous
import jax
import jax.numpy as jnp
from jax import lax
from jax.experimental import pallas as pl
from jax.experimental.pallas import tpu as pltpu
from jax.experimental.pallas import tpu_sc as plsc

NC, NS = 2, 16
CHUNK = 32
GROUP = 128
RG = 2
PAD_ROWS = 64
EPS = 1e-5
BN = 1000


def _mesh():
    return plsc.VectorSubcoreMesh(core_axis_name="c", subcore_axis_name="s",
                                  num_cores=NC, num_subcores=NS)


def _make_agg(n_chunks, n_acc, n_out, e_pad, with_count=False):
    per_core = n_chunks // NC
    g_total = e_pad // GROUP
    g_tile = g_total // NS
    nwin = g_tile // RG
    z_sl = n_out // NS

    def chunk_pass(tbl, out, src2, dst2, zeros, acc, sb, db, rb,
                   gsem, ssem, s, cnt_refs):
        pltpu.sync_copy(zeros, acc.at[pl.ds(s * z_sl, z_sl)])
        if cnt_refs is not None:
            acc_cnt, ones, zeros1, cnt_out = cnt_refs
            for k in range(GROUP // 16):
                ones[pl.ds(k * 16, 16)] = jnp.ones((16,), jnp.float32)
            pltpu.sync_copy(zeros1, acc_cnt.at[pl.ds(s * z_sl, z_sl)])
        plsc.subcore_barrier()
        g0 = s * g_tile

        def load_and_gather(w, b):
            g = g0 + w * RG
            pltpu.sync_copy(src2.at[pl.ds(g, RG)], sb[b])
            pltpu.sync_copy(dst2.at[pl.ds(g, RG)], db[b])
            for r in range(RG):
                pltpu.async_copy(tbl.at[sb[b].at[r]],
                                 rb[b].at[pl.ds(r * GROUP, GROUP)], gsem[b])

        def step(w, b, wp):
            for r in range(RG):
                pltpu.make_async_copy(
                    tbl.at[sb[b].at[r]],
                    rb[b].at[pl.ds(r * GROUP, GROUP)], gsem[b]).wait()
            for r in range(RG):
                pltpu.async_copy(rb[b].at[pl.ds(r * GROUP, GROUP)],
                                 acc.at[db[b].at[r]], ssem[b], add=True)
                if cnt_refs is not None:
                    pltpu.async_copy(ones, acc_cnt.at[db[b].at[r]],
                                     ssem[b], add=True)
            for r in range(RG):
                pltpu.make_async_copy(
                    rb[b].at[pl.ds(r * GROUP, GROUP)],
                    acc.at[db[b].at[r]], ssem[b]).wait()
                if cnt_refs is not None:
                    pltpu.make_async_copy(ones, acc_cnt.at[db[b].at[r]],
                                          ssem[b]).wait()
            pl.when(wp < nwin // 2 - 1)(lambda: load_and_gather(w + 2, b))

        load_and_gather(0, 0)
        load_and_gather(1, 1)

        def pair(wp, carry):
            step(2 * wp, 0, wp)
            step(2 * wp + 1, 1, wp)
            return carry

        lax.fori_loop(0, nwin // 2, pair, 0)
        plsc.subcore_barrier()
        pltpu.sync_copy(acc.at[pl.ds(s * z_sl, z_sl)],
                        out.at[pl.ds(s * z_sl, z_sl)])
        if cnt_refs is not None:
            pltpu.sync_copy(acc_cnt.at[pl.ds(s * z_sl, z_sl)],
                            cnt_out.at[pl.ds(s * z_sl, z_sl)])
        plsc.subcore_barrier()

    n_in = n_chunks + (4 if with_count else 3)

    def body(*refs):
        tbls = refs[:n_chunks]
        src2, dst2, zeros = refs[n_chunks:n_chunks + 3]
        zeros1 = refs[n_chunks + 3] if with_count else None
        outs = refs[n_in:n_in + n_chunks]
        cnt_out = refs[n_in + n_chunks] if with_count else None
        sc = refs[n_in + n_chunks + (1 if with_count else 0):]
        acc = sc[0]
        sb, db, rb = sc[1:3], sc[3:5], sc[5:7]
        gsem, ssem = sc[7:9], sc[9:11]
        acc_cnt = sc[11] if with_count else None
        ones = sc[12] if with_count else None
        c = lax.axis_index("c")
        s = lax.axis_index("s")
        for cc in range(NC):
            def run(cc=cc):
                for p in range(per_core):
                    j = cc * per_core + p
                    cr = None
                    if with_count and cc == 0 and p == 0:
                        cr = (acc_cnt, ones, zeros1, cnt_out)
                    chunk_pass(tbls[j], outs[j], src2, dst2, zeros,
                               acc, sb, db, rb, gsem, ssem, s, cr)
            pl.when(c == cc)(run)

    out_type = tuple(jax.ShapeDtypeStruct((n_out, CHUNK), jnp.float32)
                     for _ in range(n_chunks))
    if with_count:
        out_type = out_type + (jax.ShapeDtypeStruct((n_out,), jnp.float32),)
    scratch = [
        pltpu.VMEM_SHARED((n_acc, CHUNK), jnp.float32),
        pltpu.VMEM((RG, GROUP), jnp.int32),
        pltpu.VMEM((RG, GROUP), jnp.int32),
        pltpu.VMEM((RG, GROUP), jnp.int32),
        pltpu.VMEM((RG, GROUP), jnp.int32),
        pltpu.VMEM((RG * GROUP, CHUNK), jnp.float32),
        pltpu.VMEM((RG * GROUP, CHUNK), jnp.float32),
        pltpu.SemaphoreType.DMA,
        pltpu.SemaphoreType.DMA,
        pltpu.SemaphoreType.DMA,
        pltpu.SemaphoreType.DMA,
    ]
    if with_count:
        scratch += [
            pltpu.VMEM_SHARED((n_acc,), jnp.float32),
            pltpu.VMEM((GROUP,), jnp.float32),
        ]
    return pl.kernel(body, out_type=out_type, mesh=_mesh(),
                     scratch_types=scratch,
                     compiler_params=pltpu.CompilerParams(
                         use_tc_tiling_on_sc=False))


def _mm_call(aggs, hcs, cnt, wl, wr, bl, n):
    nb = n // BN
    na, nh = len(aggs), len(hcs)

    def kern(*refs):
        agg_r = refs[:na]
        h_r = refs[na:na + nh]
        cnt_r, wl_r, wr_r, bl_r, z_r, st_r = refs[na + nh:]
        a = jnp.concatenate([r[...] for r in agg_r], axis=1)
        h = jnp.concatenate([r[...] for r in h_r], axis=1)
        cb = cnt_r[...]
        inv = 1.0 / jnp.maximum(cb[:, 0], 1.0)
        z = (jnp.dot(a, wl_r[...], preferred_element_type=jnp.float32)
             * inv[:, None]
             + jnp.dot(h, wr_r[...], preferred_element_type=jnp.float32)
             + bl_r[...])
        z_r[...] = z
        st_r[...] = jnp.stack([jnp.sum(z, axis=0),
                               jnp.sum(z * z, axis=0)])[None]

    d = wl.shape[0]
    in_specs = (
        [pl.BlockSpec((BN, CHUNK), lambda i: (i, 0)) for _ in range(na)]
        + [pl.BlockSpec((BN, CHUNK), lambda i: (i, 0)) for _ in range(nh)]
        + [pl.BlockSpec((BN, 1), lambda i: (i, 0)),
           pl.BlockSpec((d, 128), lambda i: (0, 0)),
           pl.BlockSpec((d, 128), lambda i: (0, 0)),
           pl.BlockSpec((1, 128), lambda i: (0, 0))]
    )
    z, st = pl.pallas_call(
        kern,
        grid=(nb,),
        in_specs=in_specs,
        out_specs=(pl.BlockSpec((BN, 128), lambda i: (i, 0)),
                   pl.BlockSpec((1, 2, 128), lambda i: (i, 0, 0))),
        out_shape=(jax.ShapeDtypeStruct((n, 128), jnp.float32),
                   jax.ShapeDtypeStruct((nb, 2, 128), jnp.float32)),
    )(*aggs, *hcs, cnt, wl, wr, bl.reshape(1, 128))
    return z, st


def _bn_call(z, st, g, b, n):
    nb = n // BN

    def kern(z_r, st_r, g_r, b_r, *outs):
        stats = st_r[...]
        mu = jnp.sum(stats[:, 0, :], axis=0) * (1.0 / n)
        ex2 = jnp.sum(stats[:, 1, :], axis=0) * (1.0 / n)
        var = ex2 - mu * mu
        h = jnp.maximum(
            g_r[...] * (z_r[...] - mu[None, :])
            / jnp.sqrt(var + EPS)[None, :] + b_r[...], 0.0)
        for j, o in enumerate(outs):
            o[...] = h[:, j * CHUNK:(j + 1) * CHUNK]

    n_ch = 128 // CHUNK
    outs = pl.pallas_call(
        kern,
        grid=(nb,),
        in_specs=[pl.BlockSpec((BN, 128), lambda i: (i, 0)),
                  pl.BlockSpec((nb, 2, 128), lambda i: (0, 0, 0)),
                  pl.BlockSpec((1, 128), lambda i: (0, 0)),
                  pl.BlockSpec((1, 128), lambda i: (0, 0))],
        out_specs=tuple(pl.BlockSpec((BN, CHUNK), lambda i: (i, 0))
                        for _ in range(n_ch)),
        out_shape=tuple(jax.ShapeDtypeStruct((n, CHUNK), jnp.float32)
                        for _ in range(n_ch)),
    )(z, st, g.reshape(1, 128), b.reshape(1, 128))
    return list(outs)


def _bn_head_call(z, st, g, b, w1, b1, w2, b2, n):
    nb = n // BN

    def kern(z_r, st_r, g_r, b_r, w1_r, b1_r, w2_r, b2_r, o_r):
        stats = st_r[...]
        mu = jnp.sum(stats[:, 0, :], axis=0) * (1.0 / n)
        ex2 = jnp.sum(stats[:, 1, :], axis=0) * (1.0 / n)
        var = ex2 - mu * mu
        h = jnp.maximum(
            g_r[...] * (z_r[...] - mu[None, :])
            / jnp.sqrt(var + EPS)[None, :] + b_r[...], 0.0)
        h1 = jnp.maximum(
            jnp.dot(h, w1_r[...], preferred_element_type=jnp.float32)
            + b1_r[...], 0.0)
        o = jnp.sum(h1 * w2_r[...], axis=1, keepdims=True) + b2_r[...]
        o_r[...] = jax.nn.sigmoid(o)

    hd = w1.shape[1]
    out = pl.pallas_call(
        kern,
        grid=(nb,),
        in_specs=[pl.BlockSpec((BN, 128), lambda i: (i, 0)),
                  pl.BlockSpec((nb, 2, 128), lambda i: (0, 0, 0)),
                  pl.BlockSpec((1, 128), lambda i: (0, 0)),
                  pl.BlockSpec((1, 128), lambda i: (0, 0)),
                  pl.BlockSpec((128, hd), lambda i: (0, 0)),
                  pl.BlockSpec((1, hd), lambda i: (0, 0)),
                  pl.BlockSpec((1, hd), lambda i: (0, 0)),
                  pl.BlockSpec((1, 1), lambda i: (0, 0))],
        out_specs=pl.BlockSpec((BN, 1), lambda i: (i, 0)),
        out_shape=jax.ShapeDtypeStruct((n, 1), jnp.float32),
    )(z, st, g.reshape(1, 128), b.reshape(1, 128), w1,
      b1.reshape(1, hd), w2.reshape(1, hd), b2.reshape(1, 1))
    return out


def kernel(x, edge_index, params):
    n, in_dim = x.shape
    e = edge_index.shape[1]
    src, dst = edge_index[0], edge_index[1]

    unit = GROUP * NS * RG * NC
    e_pad = ((e + unit - 1) // unit) * unit
    pad = e_pad - e
    z_sl = ((-(-n // NS) + 127) // 128) * 128
    n_out = z_sl * NS
    n_acc = max(n_out, n + PAD_ROWS)

    ar = jnp.arange(pad, dtype=jnp.int32)
    src2 = jnp.concatenate([src, ar % n]).reshape(-1, GROUP)
    dst2 = jnp.concatenate([dst, n + (ar % PAD_ROWS)]).reshape(-1, GROUP)
    zeros32 = jnp.zeros((z_sl, CHUNK), jnp.float32)
    zeros1 = jnp.zeros((z_sl,), jnp.float32)

    agg2 = _make_agg(2, n_acc, n_out, e_pad, with_count=True)
    agg4 = _make_agg(4, n_acc, n_out, e_pad)

    hc = [lax.slice(x, (0, j * CHUNK), (n, (j + 1) * CHUNK))
          for j in range(in_dim // CHUNK)]
    cnt = None
    for i in range(3):
        if i == 0:
            *aggs, cnt_v = agg2(*hc, src2, dst2, zeros32, zeros1)
            cnt = cnt_v.reshape(n_out, 1)
        else:
            aggs = agg4(*hc, src2, dst2, zeros32)
        z, st = _mm_call(list(aggs), hc, cnt, params['Wl%d' % i],
                         params['Wr%d' % i], params['bl%d' % i], n)
        if i < 2:
            hc = _bn_call(z, st, params['bn_g%d' % i],
                          params['bn_b%d' % i], n)
        else:
            out = _bn_head_call(z, st, params['bn_g%d' % i],
                                params['bn_b%d' % i], params['fc1_W'],
                                params['fc1_b'], params['fc2_W'],
                                params['fc2_b'], n)
    return out


# speedup vs baseline: 5.8200x; 1.2117x over previous
import jax
import jax.numpy as jnp
from jax import lax
from jax.experimental import pallas as pl
from jax.experimental.pallas import tpu as pltpu
from jax.experimental.pallas import tpu_sc as plsc

NC, NS = 2, 16
CHUNK = 32
GROUP = 128
RG = 2
PAD_ROWS = 64
EPS = 1e-5
BN = 1000


def _mesh():
    return plsc.VectorSubcoreMesh(core_axis_name="c", subcore_axis_name="s",
                                  num_cores=NC, num_subcores=NS)


def _make_agg(n_chunks, n_acc, n_out, e_pad, with_count=False):
    per_core = n_chunks // NC
    g_total = e_pad // GROUP
    g_tile = g_total // NS
    nwin = g_tile // RG
    z_sl = n_out // NS

    def chunk_pass(j, tbl, out, src2, dst2, zeros, acc, sb, sb4, db, rb,
                   gsem, ssem, s, cnt_refs):
        pltpu.sync_copy(zeros, acc.at[pl.ds(s * z_sl, z_sl)])
        if cnt_refs is not None:
            acc_cnt, ones, zeros1, cnt_out = cnt_refs
            for k in range(GROUP // 16):
                ones[pl.ds(k * 16, 16)] = jnp.ones((16,), jnp.float32)
            pltpu.sync_copy(zeros1, acc_cnt.at[pl.ds(s * z_sl, z_sl)])
        plsc.subcore_barrier()
        g0 = s * g_tile

        def load_and_gather(w, b):
            g = g0 + w * RG
            pltpu.sync_copy(src2.at[pl.ds(g, RG)], sb[b])
            pltpu.sync_copy(dst2.at[pl.ds(g, RG)], db[b])
            for r in range(RG):
                for k in range(GROUP // 16):
                    sb4[b][r, pl.ds(k * 16, 16)] = (
                        sb[b][r, pl.ds(k * 16, 16)] * n_chunks + j)
                pltpu.async_copy(tbl.at[sb4[b].at[r]],
                                 rb[b].at[pl.ds(r * GROUP, GROUP)], gsem[b])

        def step(w, b, wp):
            for r in range(RG):
                pltpu.make_async_copy(
                    tbl.at[sb4[b].at[r]],
                    rb[b].at[pl.ds(r * GROUP, GROUP)], gsem[b]).wait()
            for r in range(RG):
                pltpu.async_copy(rb[b].at[pl.ds(r * GROUP, GROUP)],
                                 acc.at[db[b].at[r]], ssem[b], add=True)
                if cnt_refs is not None:
                    pltpu.async_copy(ones, acc_cnt.at[db[b].at[r]],
                                     ssem[b], add=True)
            for r in range(RG):
                pltpu.make_async_copy(
                    rb[b].at[pl.ds(r * GROUP, GROUP)],
                    acc.at[db[b].at[r]], ssem[b]).wait()
                if cnt_refs is not None:
                    pltpu.make_async_copy(ones, acc_cnt.at[db[b].at[r]],
                                          ssem[b]).wait()
            pl.when(wp < nwin // 2 - 1)(lambda: load_and_gather(w + 2, b))

        load_and_gather(0, 0)
        load_and_gather(1, 1)

        def pair(wp, carry):
            step(2 * wp, 0, wp)
            step(2 * wp + 1, 1, wp)
            return carry

        lax.fori_loop(0, nwin // 2, pair, 0)
        plsc.subcore_barrier()
        pltpu.sync_copy(acc.at[pl.ds(s * z_sl, z_sl)],
                        out.at[pl.ds(s * z_sl, z_sl),
                               pl.ds(j * CHUNK, CHUNK)])
        if cnt_refs is not None:
            pltpu.sync_copy(acc_cnt.at[pl.ds(s * z_sl, z_sl)],
                            cnt_out.at[pl.ds(s * z_sl, z_sl)])
        plsc.subcore_barrier()

    n_in = 5 if with_count else 4

    def body(*refs):
        tbl = refs[0]
        src2, dst2, zeros = refs[1:4]
        zeros1 = refs[4] if with_count else None
        out = refs[n_in]
        cnt_out = refs[n_in + 1] if with_count else None
        sc = refs[n_in + (2 if with_count else 1):]
        acc = sc[0]
        sb, sb4, db = sc[1:3], sc[3:5], sc[5:7]
        rb = sc[7:9]
        gsem, ssem = sc[9:11], sc[11:13]
        acc_cnt = sc[13] if with_count else None
        ones = sc[14] if with_count else None
        c = lax.axis_index("c")
        s = lax.axis_index("s")
        for cc in range(NC):
            def run(cc=cc):
                for p in range(per_core):
                    j = cc * per_core + p
                    cr = None
                    if with_count and cc == 0 and p == 0:
                        cr = (acc_cnt, ones, zeros1, cnt_out)
                    chunk_pass(j, tbl, out, src2, dst2, zeros,
                               acc, sb, sb4, db, rb, gsem, ssem, s, cr)
            pl.when(c == cc)(run)

    out_type = (jax.ShapeDtypeStruct((n_out, n_chunks * CHUNK),
                                     jnp.float32),)
    if with_count:
        out_type = out_type + (jax.ShapeDtypeStruct((n_out,), jnp.float32),)
    scratch = (
        [pltpu.VMEM_SHARED((n_acc, CHUNK), jnp.float32)]
        + [pltpu.VMEM((RG, GROUP), jnp.int32) for _ in range(6)]
        + [pltpu.VMEM((RG * GROUP, CHUNK), jnp.float32) for _ in range(2)]
        + [pltpu.SemaphoreType.DMA for _ in range(4)]
    )
    if with_count:
        scratch += [
            pltpu.VMEM_SHARED((n_acc,), jnp.float32),
            pltpu.VMEM((GROUP,), jnp.float32),
        ]
    return pl.kernel(body, out_type=out_type, mesh=_mesh(),
                     scratch_types=scratch,
                     compiler_params=pltpu.CompilerParams(
                         use_tc_tiling_on_sc=False))


def _mm_call(agg, h, cnt, wl, wr, bl, n):
    nb = n // BN
    d = wl.shape[0]

    def kern(agg_r, h_r, cnt_r, wl_r, wr_r, bl_r, z_r, st_r):
        inv = 1.0 / jnp.maximum(cnt_r[...][:, 0], 1.0)
        z = (jnp.dot(agg_r[...], wl_r[...],
                     preferred_element_type=jnp.float32) * inv[:, None]
             + jnp.dot(h_r[...], wr_r[...],
                       preferred_element_type=jnp.float32)
             + bl_r[...])
        z_r[...] = z
        st_r[...] = jnp.stack([jnp.sum(z, axis=0),
                               jnp.sum(z * z, axis=0)])[None]

    z, st = pl.pallas_call(
        kern,
        grid=(nb,),
        in_specs=[pl.BlockSpec((BN, d), lambda i: (i, 0)),
                  pl.BlockSpec((BN, d), lambda i: (i, 0)),
                  pl.BlockSpec((BN, 1), lambda i: (i, 0)),
                  pl.BlockSpec((d, 128), lambda i: (0, 0)),
                  pl.BlockSpec((d, 128), lambda i: (0, 0)),
                  pl.BlockSpec((1, 128), lambda i: (0, 0))],
        out_specs=(pl.BlockSpec((BN, 128), lambda i: (i, 0)),
                   pl.BlockSpec((1, 2, 128), lambda i: (i, 0, 0))),
        out_shape=(jax.ShapeDtypeStruct((n, 128), jnp.float32),
                   jax.ShapeDtypeStruct((nb, 2, 128), jnp.float32)),
    )(agg, h, cnt, wl, wr, bl.reshape(1, 128))
    return z, st


def _bn_call(z, st, g, b, n):
    nb = n // BN

    def kern(z_r, st_r, g_r, b_r, o_r):
        stats = st_r[...]
        mu = jnp.sum(stats[:, 0, :], axis=0) * (1.0 / n)
        ex2 = jnp.sum(stats[:, 1, :], axis=0) * (1.0 / n)
        var = ex2 - mu * mu
        o_r[...] = jnp.maximum(
            g_r[...] * (z_r[...] - mu[None, :])
            / jnp.sqrt(var + EPS)[None, :] + b_r[...], 0.0)

    return pl.pallas_call(
        kern,
        grid=(nb,),
        in_specs=[pl.BlockSpec((BN, 128), lambda i: (i, 0)),
                  pl.BlockSpec((nb, 2, 128), lambda i: (0, 0, 0)),
                  pl.BlockSpec((1, 128), lambda i: (0, 0)),
                  pl.BlockSpec((1, 128), lambda i: (0, 0))],
        out_specs=pl.BlockSpec((BN, 128), lambda i: (i, 0)),
        out_shape=jax.ShapeDtypeStruct((n, 128), jnp.float32),
    )(z, st, g.reshape(1, 128), b.reshape(1, 128))


def _bn_head_call(z, st, g, b, w1, b1, w2, b2, n):
    nb = n // BN
    hd = w1.shape[1]

    def kern(z_r, st_r, g_r, b_r, w1_r, b1_r, w2_r, b2_r, o_r):
        stats = st_r[...]
        mu = jnp.sum(stats[:, 0, :], axis=0) * (1.0 / n)
        ex2 = jnp.sum(stats[:, 1, :], axis=0) * (1.0 / n)
        var = ex2 - mu * mu
        h = jnp.maximum(
            g_r[...] * (z_r[...] - mu[None, :])
            / jnp.sqrt(var + EPS)[None, :] + b_r[...], 0.0)
        h1 = jnp.maximum(
            jnp.dot(h, w1_r[...], preferred_element_type=jnp.float32)
            + b1_r[...], 0.0)
        o = jnp.sum(h1 * w2_r[...], axis=1, keepdims=True) + b2_r[...]
        o_r[...] = jax.nn.sigmoid(o)

    return pl.pallas_call(
        kern,
        grid=(nb,),
        in_specs=[pl.BlockSpec((BN, 128), lambda i: (i, 0)),
                  pl.BlockSpec((nb, 2, 128), lambda i: (0, 0, 0)),
                  pl.BlockSpec((1, 128), lambda i: (0, 0)),
                  pl.BlockSpec((1, 128), lambda i: (0, 0)),
                  pl.BlockSpec((128, hd), lambda i: (0, 0)),
                  pl.BlockSpec((1, hd), lambda i: (0, 0)),
                  pl.BlockSpec((1, hd), lambda i: (0, 0)),
                  pl.BlockSpec((1, 1), lambda i: (0, 0))],
        out_specs=pl.BlockSpec((BN, 1), lambda i: (i, 0)),
        out_shape=jax.ShapeDtypeStruct((n, 1), jnp.float32),
    )(z, st, g.reshape(1, 128), b.reshape(1, 128), w1,
      b1.reshape(1, hd), w2.reshape(1, hd), b2.reshape(1, 1))


def kernel(x, edge_index, params):
    n, in_dim = x.shape
    e = edge_index.shape[1]
    src, dst = edge_index[0], edge_index[1]

    unit = GROUP * NS * RG * NC
    e_pad = ((e + unit - 1) // unit) * unit
    pad = e_pad - e
    z_sl = ((-(-n // NS) + 127) // 128) * 128
    n_out = z_sl * NS
    n_acc = max(n_out, n + PAD_ROWS)

    ar = jnp.arange(pad, dtype=jnp.int32)
    src2 = jnp.concatenate([src, ar % n]).reshape(-1, GROUP)
    dst2 = jnp.concatenate([dst, n + (ar % PAD_ROWS)]).reshape(-1, GROUP)
    zeros32 = jnp.zeros((z_sl, CHUNK), jnp.float32)
    zeros1 = jnp.zeros((z_sl,), jnp.float32)

    agg2 = _make_agg(2, n_acc, n_out, e_pad, with_count=True)
    agg4 = _make_agg(4, n_acc, n_out, e_pad)

    h = x
    cnt = None
    for i in range(3):
        nch = h.shape[1] // CHUNK
        tbl = h.reshape(n * nch, CHUNK)
        if i == 0:
            agg, cnt_v = agg2(tbl, src2, dst2, zeros32, zeros1)
            cnt = cnt_v.reshape(n_out, 1)
        else:
            (agg,) = agg4(tbl, src2, dst2, zeros32)
        z, st = _mm_call(agg, h, cnt, params['Wl%d' % i],
                         params['Wr%d' % i], params['bl%d' % i], n)
        if i < 2:
            h = _bn_call(z, st, params['bn_g%d' % i],
                         params['bn_b%d' % i], n)
        else:
            out = _bn_head_call(z, st, params['bn_g%d' % i],
                                params['bn_b%d' % i], params['fc1_W'],
                                params['fc1_b'], params['fc2_W'],
                                params['fc2_b'], n)
    return out


# speedup vs baseline: 9.4588x; 1.6252x over previous
import jax
import jax.numpy as jnp
from jax import lax
from jax.experimental import pallas as pl
from jax.experimental.pallas import tpu as pltpu
from jax.experimental.pallas import tpu_sc as plsc

NC, NS = 2, 16
CHUNK = 32
GROUP = 128
SUPER = 8
PAD_ROWS = 64
EPS = 1e-5
BN = 1000


def _mesh():
    return plsc.VectorSubcoreMesh(core_axis_name="c", subcore_axis_name="s",
                                  num_cores=NC, num_subcores=NS)


def _make_agg(n_chunks, n_acc, n_out, e_pad, with_count=False):
    per_core = n_chunks // NC
    g_total = e_pad // GROUP
    g_tile = g_total // NS
    nsb = g_tile // SUPER
    z_sl = n_out // NS

    def chunk_pass(j, tbl, out, src2, dst2, zeros, acc, si, s4, di, rb,
                   gsem, ssem, isem, s, cnt_refs):
        pltpu.sync_copy(zeros, acc.at[pl.ds(s * z_sl, z_sl)])
        if cnt_refs is not None:
            acc_cnt, ones, zeros1, cnt_out = cnt_refs
            for k in range(GROUP // 16):
                ones[pl.ds(k * 16, 16)] = jnp.ones((16,), jnp.float32)
            pltpu.sync_copy(zeros1, acc_cnt.at[pl.ds(s * z_sl, z_sl)])
        plsc.subcore_barrier()
        g0 = s * g_tile

        def load_idx(sbk, sl):
            g = g0 + sbk * SUPER
            pltpu.async_copy(src2.at[pl.ds(g, SUPER)], si[sl], isem)
            pltpu.async_copy(dst2.at[pl.ds(g, SUPER)], di[sl], isem)

        def wait_idx(sbk, sl):
            g = g0 + sbk * SUPER
            pltpu.make_async_copy(src2.at[pl.ds(g, SUPER)], si[sl],
                                  isem).wait()
            pltpu.make_async_copy(dst2.at[pl.ds(g, SUPER)], di[sl],
                                  isem).wait()

        def calc_s4(sl):
            for gi in range(SUPER):
                for k in range(GROUP // 16):
                    s4[sl][gi, pl.ds(k * 16, 16)] = (
                        si[sl][gi, pl.ds(k * 16, 16)] * n_chunks + j)

        def gissue(sbk, gi, isl, rsl):
            pltpu.async_copy(tbl.at[s4[isl].at[gi]], rb[rsl], gsem[rsl])

        def gwait(isl, gi, rsl):
            pltpu.make_async_copy(tbl.at[s4[isl].at[gi]], rb[rsl],
                                  gsem[rsl]).wait()

        def sissue(isl, gi, rsl):
            pltpu.async_copy(rb[rsl], acc.at[di[isl].at[gi]],
                             ssem[rsl], add=True)
            if cnt_refs is not None:
                pltpu.async_copy(ones, acc_cnt.at[di[isl].at[gi]],
                                 ssem[rsl], add=True)

        def swait(rsl):
            pltpu.make_async_copy(rb[rsl], acc.at[pl.ds(0, GROUP)],
                                  ssem[rsl]).wait()
            if cnt_refs is not None:
                pltpu.make_async_copy(ones, acc_cnt.at[pl.ds(0, GROUP)],
                                      ssem[rsl]).wait()

        load_idx(0, 0)
        wait_idx(0, 0)
        calc_s4(0)
        for gg in range(3):
            gissue(0, gg, 0, gg)

        def sblock(sbk, carry):
            isl_d = lax.rem(sbk, 2)

            def do(cur_par):
                isl = cur_par
                nxt = 1 - cur_par
                for gi in range(SUPER):
                    rsl = gi % 4
                    gwait(isl, gi, rsl)
                    sissue(isl, gi, rsl)
                    nsl = (gi + 3) % 4
                    if gi == 0:
                        pl.when(sbk > 0)(lambda: swait(nsl))
                        pl.when(sbk < nsb - 1)(
                            lambda: load_idx(sbk + 1, nxt))
                    else:
                        swait(nsl)
                    if gi == 4:
                        def prep():
                            wait_idx(sbk + 1, nxt)
                            calc_s4(nxt)
                        pl.when(sbk < nsb - 1)(prep)
                    if gi < 5:
                        gissue(sbk, gi + 3, isl, nsl)
                    else:
                        pl.when(sbk < nsb - 1)(
                            lambda gi=gi, nsl=nsl:
                            gissue(sbk + 1, gi - 5, nxt, nsl))

            for par in range(2):
                pl.when(isl_d == par)(lambda par=par: do(par))
            return carry

        lax.fori_loop(0, nsb, sblock, 0)
        swait((g_tile - 1) % 4)
        plsc.subcore_barrier()
        pltpu.sync_copy(acc.at[pl.ds(s * z_sl, z_sl)],
                        out.at[pl.ds(s * z_sl, z_sl),
                               pl.ds(j * CHUNK, CHUNK)])
        if cnt_refs is not None:
            pltpu.sync_copy(acc_cnt.at[pl.ds(s * z_sl, z_sl)],
                            cnt_out.at[pl.ds(s * z_sl, z_sl)])
        plsc.subcore_barrier()

    n_in = 5 if with_count else 4

    def body(*refs):
        tbl = refs[0]
        src2, dst2, zeros = refs[1:4]
        zeros1 = refs[4] if with_count else None
        out = refs[n_in]
        cnt_out = refs[n_in + 1] if with_count else None
        sc = refs[n_in + (2 if with_count else 1):]
        acc = sc[0]
        si, s4, di = sc[1:3], sc[3:5], sc[5:7]
        rb = sc[7:11]
        gsem, ssem = sc[11:15], sc[15:19]
        isem = sc[19]
        acc_cnt = sc[20] if with_count else None
        ones = sc[21] if with_count else None
        c = lax.axis_index("c")
        s = lax.axis_index("s")
        for cc in range(NC):
            def run(cc=cc):
                for p in range(per_core):
                    j = cc * per_core + p
                    cr = None
                    if with_count and cc == 0 and p == 0:
                        cr = (acc_cnt, ones, zeros1, cnt_out)
                    chunk_pass(j, tbl, out, src2, dst2, zeros,
                               acc, si, s4, di, rb, gsem, ssem, isem, s, cr)
            pl.when(c == cc)(run)

    out_type = (jax.ShapeDtypeStruct((n_out, n_chunks * CHUNK),
                                     jnp.float32),)
    if with_count:
        out_type = out_type + (jax.ShapeDtypeStruct((n_out,), jnp.float32),)
    scratch = (
        [pltpu.VMEM_SHARED((n_acc, CHUNK), jnp.float32)]
        + [pltpu.VMEM((SUPER, GROUP), jnp.int32) for _ in range(6)]
        + [pltpu.VMEM((GROUP, CHUNK), jnp.float32) for _ in range(4)]
        + [pltpu.SemaphoreType.DMA for _ in range(9)]
    )
    if with_count:
        scratch += [
            pltpu.VMEM_SHARED((n_acc,), jnp.float32),
            pltpu.VMEM((GROUP,), jnp.float32),
        ]
    return pl.kernel(body, out_type=out_type, mesh=_mesh(),
                     scratch_types=scratch,
                     compiler_params=pltpu.CompilerParams(
                         use_tc_tiling_on_sc=False))


def _mm_call(agg, h, cnt, wl, wr, bl, n):
    nb = n // BN
    d = wl.shape[0]

    def kern(agg_r, h_r, cnt_r, wl_r, wr_r, bl_r, z_r, st_r):
        inv = 1.0 / jnp.maximum(cnt_r[...][:, 0], 1.0)
        z = (jnp.dot(agg_r[...], wl_r[...],
                     preferred_element_type=jnp.float32) * inv[:, None]
             + jnp.dot(h_r[...], wr_r[...],
                       preferred_element_type=jnp.float32)
             + bl_r[...])
        z_r[...] = z
        st_r[...] = jnp.stack([jnp.sum(z, axis=0),
                               jnp.sum(z * z, axis=0)])[None]

    z, st = pl.pallas_call(
        kern,
        grid=(nb,),
        in_specs=[pl.BlockSpec((BN, d), lambda i: (i, 0)),
                  pl.BlockSpec((BN, d), lambda i: (i, 0)),
                  pl.BlockSpec((BN, 1), lambda i: (i, 0)),
                  pl.BlockSpec((d, 128), lambda i: (0, 0)),
                  pl.BlockSpec((d, 128), lambda i: (0, 0)),
                  pl.BlockSpec((1, 128), lambda i: (0, 0))],
        out_specs=(pl.BlockSpec((BN, 128), lambda i: (i, 0)),
                   pl.BlockSpec((1, 2, 128), lambda i: (i, 0, 0))),
        out_shape=(jax.ShapeDtypeStruct((n, 128), jnp.float32),
                   jax.ShapeDtypeStruct((nb, 2, 128), jnp.float32)),
    )(agg, h, cnt, wl, wr, bl.reshape(1, 128))
    return z, st


def _bn_call(z, st, g, b, n):
    nb = n // BN

    def kern(z_r, st_r, g_r, b_r, o_r):
        stats = st_r[...]
        mu = jnp.sum(stats[:, 0, :], axis=0) * (1.0 / n)
        ex2 = jnp.sum(stats[:, 1, :], axis=0) * (1.0 / n)
        var = ex2 - mu * mu
        o_r[...] = jnp.maximum(
            g_r[...] * (z_r[...] - mu[None, :])
            / jnp.sqrt(var + EPS)[None, :] + b_r[...], 0.0)

    return pl.pallas_call(
        kern,
        grid=(nb,),
        in_specs=[pl.BlockSpec((BN, 128), lambda i: (i, 0)),
                  pl.BlockSpec((nb, 2, 128), lambda i: (0, 0, 0)),
                  pl.BlockSpec((1, 128), lambda i: (0, 0)),
                  pl.BlockSpec((1, 128), lambda i: (0, 0))],
        out_specs=pl.BlockSpec((BN, 128), lambda i: (i, 0)),
        out_shape=jax.ShapeDtypeStruct((n, 128), jnp.float32),
    )(z, st, g.reshape(1, 128), b.reshape(1, 128))


def _bn_head_call(z, st, g, b, w1, b1, w2, b2, n):
    nb = n // BN
    hd = w1.shape[1]

    def kern(z_r, st_r, g_r, b_r, w1_r, b1_r, w2_r, b2_r, o_r):
        stats = st_r[...]
        mu = jnp.sum(stats[:, 0, :], axis=0) * (1.0 / n)
        ex2 = jnp.sum(stats[:, 1, :], axis=0) * (1.0 / n)
        var = ex2 - mu * mu
        h = jnp.maximum(
            g_r[...] * (z_r[...] - mu[None, :])
            / jnp.sqrt(var + EPS)[None, :] + b_r[...], 0.0)
        h1 = jnp.maximum(
            jnp.dot(h, w1_r[...], preferred_element_type=jnp.float32)
            + b1_r[...], 0.0)
        o = jnp.sum(h1 * w2_r[...], axis=1, keepdims=True) + b2_r[...]
        o_r[...] = jax.nn.sigmoid(o)

    return pl.pallas_call(
        kern,
        grid=(nb,),
        in_specs=[pl.BlockSpec((BN, 128), lambda i: (i, 0)),
                  pl.BlockSpec((nb, 2, 128), lambda i: (0, 0, 0)),
                  pl.BlockSpec((1, 128), lambda i: (0, 0)),
                  pl.BlockSpec((1, 128), lambda i: (0, 0)),
                  pl.BlockSpec((128, hd), lambda i: (0, 0)),
                  pl.BlockSpec((1, hd), lambda i: (0, 0)),
                  pl.BlockSpec((1, hd), lambda i: (0, 0)),
                  pl.BlockSpec((1, 1), lambda i: (0, 0))],
        out_specs=pl.BlockSpec((BN, 1), lambda i: (i, 0)),
        out_shape=jax.ShapeDtypeStruct((n, 1), jnp.float32),
    )(z, st, g.reshape(1, 128), b.reshape(1, 128), w1,
      b1.reshape(1, hd), w2.reshape(1, hd), b2.reshape(1, 1))


def kernel(x, edge_index, params):
    n, in_dim = x.shape
    e = edge_index.shape[1]
    src, dst = edge_index[0], edge_index[1]

    unit = GROUP * NS * SUPER
    e_pad = ((e + unit - 1) // unit) * unit
    pad = e_pad - e
    z_sl = ((-(-n // NS) + 127) // 128) * 128
    n_out = z_sl * NS
    n_acc = max(n_out, n + PAD_ROWS)

    ar = jnp.arange(pad, dtype=jnp.int32)
    src2 = jnp.concatenate([src, ar % n]).reshape(-1, GROUP)
    dst2 = jnp.concatenate([dst, n + (ar % PAD_ROWS)]).reshape(-1, GROUP)
    zeros32 = jnp.zeros((z_sl, CHUNK), jnp.float32)
    zeros1 = jnp.zeros((z_sl,), jnp.float32)

    agg2 = _make_agg(2, n_acc, n_out, e_pad, with_count=True)
    agg4 = _make_agg(4, n_acc, n_out, e_pad)

    h = x
    cnt = None
    for i in range(3):
        nch = h.shape[1] // CHUNK
        tbl = h.reshape(n * nch, CHUNK)
        if i == 0:
            agg, cnt_v = agg2(tbl, src2, dst2, zeros32, zeros1)
            cnt = cnt_v.reshape(n_out, 1)
        else:
            (agg,) = agg4(tbl, src2, dst2, zeros32)
        z, st = _mm_call(agg, h, cnt, params['Wl%d' % i],
                         params['Wr%d' % i], params['bl%d' % i], n)
        if i < 2:
            h = _bn_call(z, st, params['bn_g%d' % i],
                         params['bn_b%d' % i], n)
        else:
            out = _bn_head_call(z, st, params['bn_g%d' % i],
                                params['bn_b%d' % i], params['fc1_W'],
                                params['fc1_b'], params['fc2_W'],
                                params['fc2_b'], n)
    return out


# speedup vs baseline: 9.9709x; 1.0541x over previous
import jax
import jax.numpy as jnp
from jax import lax
from jax.experimental import pallas as pl
from jax.experimental.pallas import tpu as pltpu
from jax.experimental.pallas import tpu_sc as plsc

NC, NS = 2, 16
CHUNK = 32
GROUP = 128
SUPER = 8
PAD_ROWS = 64
EPS = 1e-5
BN = 1000


def _mesh():
    return plsc.VectorSubcoreMesh(core_axis_name="c", subcore_axis_name="s",
                                  num_cores=NC, num_subcores=NS)


def _make_agg(n_chunks, n_acc, n_out, e_pad, with_count=False):
    per_core = n_chunks // NC
    g_total = e_pad // GROUP
    g_tile = g_total // NS
    nsb = g_tile // SUPER
    z_sl = n_out // NS

    def chunk_pass(j, tbl, out, src2, dst2, zeros, acc, si, s4, di, rb,
                   gsem, ssem, isem, s, cnt_refs):
        pltpu.sync_copy(zeros, acc.at[pl.ds(s * z_sl, z_sl)])
        if cnt_refs is not None:
            acc_cnt, ones, zeros1, cnt_out = cnt_refs
            for k in range(GROUP // 16):
                ones[pl.ds(k * 16, 16)] = jnp.ones((16,), jnp.float32)
            pltpu.sync_copy(zeros1, acc_cnt.at[pl.ds(s * z_sl, z_sl)])
        plsc.subcore_barrier()
        g0 = s * g_tile

        def load_idx(sbk, sl):
            g = g0 + sbk * SUPER
            pltpu.async_copy(src2.at[pl.ds(g, SUPER)], si[sl], isem)
            pltpu.async_copy(dst2.at[pl.ds(g, SUPER)], di[sl], isem)

        def wait_idx(sbk, sl):
            g = g0 + sbk * SUPER
            pltpu.make_async_copy(src2.at[pl.ds(g, SUPER)], si[sl],
                                  isem).wait()
            pltpu.make_async_copy(dst2.at[pl.ds(g, SUPER)], di[sl],
                                  isem).wait()

        def calc_s4(sl):
            for gi in range(SUPER):
                for k in range(GROUP // 16):
                    s4[sl][gi, pl.ds(k * 16, 16)] = (
                        si[sl][gi, pl.ds(k * 16, 16)] * n_chunks + j)

        def gissue(sbk, gi, isl, rsl):
            pltpu.async_copy(tbl.at[s4[isl].at[gi]], rb[rsl], gsem[rsl])

        def gwait(isl, gi, rsl):
            pltpu.make_async_copy(tbl.at[s4[isl].at[gi]], rb[rsl],
                                  gsem[rsl]).wait()

        def sissue(isl, gi, rsl):
            pltpu.async_copy(rb[rsl], acc.at[di[isl].at[gi]],
                             ssem[rsl], add=True)
            if cnt_refs is not None:
                pltpu.async_copy(ones, acc_cnt.at[di[isl].at[gi]],
                                 ssem[rsl], add=True)

        def swait(rsl):
            pltpu.make_async_copy(rb[rsl], acc.at[pl.ds(0, GROUP)],
                                  ssem[rsl]).wait()
            if cnt_refs is not None:
                pltpu.make_async_copy(ones, acc_cnt.at[pl.ds(0, GROUP)],
                                      ssem[rsl]).wait()

        load_idx(0, 0)
        wait_idx(0, 0)
        calc_s4(0)
        for gg in range(3):
            gissue(0, gg, 0, gg)

        def sblock(sbk, carry):
            isl_d = lax.rem(sbk, 2)

            def do(cur_par):
                isl = cur_par
                nxt = 1 - cur_par
                for gi in range(SUPER):
                    rsl = gi % 4
                    gwait(isl, gi, rsl)
                    sissue(isl, gi, rsl)
                    nsl = (gi + 3) % 4
                    if gi == 0:
                        pl.when(sbk > 0)(lambda: swait(nsl))
                        pl.when(sbk < nsb - 1)(
                            lambda: load_idx(sbk + 1, nxt))
                    else:
                        swait(nsl)
                    if gi == 4:
                        def prep():
                            wait_idx(sbk + 1, nxt)
                            calc_s4(nxt)
                        pl.when(sbk < nsb - 1)(prep)
                    if gi < 5:
                        gissue(sbk, gi + 3, isl, nsl)
                    else:
                        pl.when(sbk < nsb - 1)(
                            lambda gi=gi, nsl=nsl:
                            gissue(sbk + 1, gi - 5, nxt, nsl))

            for par in range(2):
                pl.when(isl_d == par)(lambda par=par: do(par))
            return carry

        lax.fori_loop(0, nsb, sblock, 0)
        swait((g_tile - 1) % 4)
        plsc.subcore_barrier()
        pltpu.sync_copy(acc.at[pl.ds(s * z_sl, z_sl)],
                        out.at[pl.ds(s * z_sl, z_sl),
                               pl.ds(j * CHUNK, CHUNK)])
        if cnt_refs is not None:
            pltpu.sync_copy(acc_cnt.at[pl.ds(s * z_sl, z_sl)],
                            cnt_out.at[pl.ds(s * z_sl, z_sl)])
        plsc.subcore_barrier()

    n_in = 5 if with_count else 4

    def body(*refs):
        tbl = refs[0]
        src2, dst2, zeros = refs[1:4]
        zeros1 = refs[4] if with_count else None
        out = refs[n_in]
        cnt_out = refs[n_in + 1] if with_count else None
        sc = refs[n_in + (2 if with_count else 1):]
        acc = sc[0]
        si, s4, di = sc[1:3], sc[3:5], sc[5:7]
        rb = sc[7:11]
        gsem, ssem = sc[11:15], sc[15:19]
        isem = sc[19]
        acc_cnt = sc[20] if with_count else None
        ones = sc[21] if with_count else None
        c = lax.axis_index("c")
        s = lax.axis_index("s")
        for cc in range(NC):
            def run(cc=cc):
                for p in range(per_core):
                    j = cc * per_core + p
                    cr = None
                    if with_count and cc == 0 and p == 0:
                        cr = (acc_cnt, ones, zeros1, cnt_out)
                    chunk_pass(j, tbl, out, src2, dst2, zeros,
                               acc, si, s4, di, rb, gsem, ssem, isem, s, cr)
            pl.when(c == cc)(run)

    out_type = (jax.ShapeDtypeStruct((n_out, n_chunks * CHUNK),
                                     jnp.float32),)
    if with_count:
        out_type = out_type + (jax.ShapeDtypeStruct((n_out,), jnp.float32),)
    scratch = (
        [pltpu.VMEM_SHARED((n_acc, CHUNK), jnp.float32)]
        + [pltpu.VMEM((SUPER, GROUP), jnp.int32) for _ in range(6)]
        + [pltpu.VMEM((GROUP, CHUNK), jnp.float32) for _ in range(4)]
        + [pltpu.SemaphoreType.DMA for _ in range(9)]
    )
    if with_count:
        scratch += [
            pltpu.VMEM_SHARED((n_acc,), jnp.float32),
            pltpu.VMEM((GROUP,), jnp.float32),
        ]
    return pl.kernel(body, out_type=out_type, mesh=_mesh(),
                     scratch_types=scratch,
                     compiler_params=pltpu.CompilerParams(
                         use_tc_tiling_on_sc=False))


def _layer_call(agg, h, cnt, wl, wr, bl, g, bb, n, head=None):
    nb = n // BN
    d = wl.shape[0]

    def common_z(agg_r, h_r, cnt_r, wl_r, wr_r, bl_r, z_scr, st_scr, i):
        @pl.when(i == 0)
        def _():
            st_scr[...] = jnp.zeros_like(st_scr)
        inv = 1.0 / jnp.maximum(cnt_r[...][:, 0], 1.0)
        z = (jnp.dot(agg_r[...], wl_r[...],
                     preferred_element_type=jnp.float32) * inv[:, None]
             + jnp.dot(h_r[...], wr_r[...],
                       preferred_element_type=jnp.float32)
             + bl_r[...])
        z_scr[pl.ds(i * BN, BN), :] = z
        st_scr[...] += jnp.stack([jnp.sum(z, axis=0),
                                  jnp.sum(z * z, axis=0)])

    def norm(z_scr, st_scr, g_r, b_r, i):
        stats = st_scr[...]
        mu = stats[0] * (1.0 / n)
        var = stats[1] * (1.0 / n) - mu * mu
        z = z_scr[pl.ds(i * BN, BN), :]
        return jnp.maximum(
            g_r[...] * (z - mu[None, :]) / jnp.sqrt(var + EPS)[None, :]
            + b_r[...], 0.0)

    if head is None:
        def kern(agg_r, h_r, cnt_r, wl_r, wr_r, bl_r, g_r, b_r,
                 h_out, z_scr, st_scr):
            p, i = pl.program_id(0), pl.program_id(1)
            pl.when(p == 0)(lambda: common_z(agg_r, h_r, cnt_r, wl_r,
                                             wr_r, bl_r, z_scr, st_scr, i))

            @pl.when(p == 1)
            def _():
                h_out[...] = norm(z_scr, st_scr, g_r, b_r, i)

        extra_in = []
        out_spec = pl.BlockSpec((BN, 128), lambda p, i: (p * i, 0))
        out_shape = jax.ShapeDtypeStruct((n, 128), jnp.float32)
        args = ()
    else:
        w1, b1, w2, b2 = head
        hd = w1.shape[1]

        def kern(agg_r, h_r, cnt_r, wl_r, wr_r, bl_r, g_r, b_r,
                 w1_r, b1_r, w2_r, b2_r, o_out, z_scr, st_scr):
            p, i = pl.program_id(0), pl.program_id(1)
            pl.when(p == 0)(lambda: common_z(agg_r, h_r, cnt_r, wl_r,
                                             wr_r, bl_r, z_scr, st_scr, i))

            @pl.when(p == 1)
            def _():
                hh = norm(z_scr, st_scr, g_r, b_r, i)
                h1 = jnp.maximum(
                    jnp.dot(hh, w1_r[...],
                            preferred_element_type=jnp.float32)
                    + b1_r[...], 0.0)
                o = (jnp.sum(h1 * w2_r[...], axis=1, keepdims=True)
                     + b2_r[...])
                o_out[...] = jax.nn.sigmoid(o)

        extra_in = [pl.BlockSpec((128, hd), lambda p, i: (0, 0)),
                    pl.BlockSpec((1, hd), lambda p, i: (0, 0)),
                    pl.BlockSpec((1, hd), lambda p, i: (0, 0)),
                    pl.BlockSpec((1, 1), lambda p, i: (0, 0))]
        out_spec = pl.BlockSpec((BN, 1), lambda p, i: (p * i, 0))
        out_shape = jax.ShapeDtypeStruct((n, 1), jnp.float32)
        args = (w1, b1.reshape(1, hd), w2.reshape(1, hd),
                b2.reshape(1, 1))

    return pl.pallas_call(
        kern,
        grid=(2, nb),
        in_specs=[pl.BlockSpec((BN, d), lambda p, i: ((1 - p) * i, 0)),
                  pl.BlockSpec((BN, d), lambda p, i: ((1 - p) * i, 0)),
                  pl.BlockSpec((BN, 1), lambda p, i: ((1 - p) * i, 0)),
                  pl.BlockSpec((d, 128), lambda p, i: (0, 0)),
                  pl.BlockSpec((d, 128), lambda p, i: (0, 0)),
                  pl.BlockSpec((1, 128), lambda p, i: (0, 0)),
                  pl.BlockSpec((1, 128), lambda p, i: (0, 0)),
                  pl.BlockSpec((1, 128), lambda p, i: (0, 0))] + extra_in,
        out_specs=out_spec,
        out_shape=out_shape,
        scratch_shapes=[pltpu.VMEM((n, 128), jnp.float32),
                        pltpu.VMEM((2, 128), jnp.float32)],
    )(agg, h, cnt, wl, wr, bl.reshape(1, 128), g.reshape(1, 128),
      bb.reshape(1, 128), *args)


def kernel(x, edge_index, params):
    n, in_dim = x.shape
    e = edge_index.shape[1]
    src, dst = edge_index[0], edge_index[1]

    unit = GROUP * NS * SUPER
    e_pad = ((e + unit - 1) // unit) * unit
    pad = e_pad - e
    z_sl = ((-(-n // NS) + 127) // 128) * 128
    n_out = z_sl * NS
    n_acc = max(n_out, n + PAD_ROWS)

    ar = jnp.arange(pad, dtype=jnp.int32)
    src2 = jnp.concatenate([src, ar % n]).reshape(-1, GROUP)
    dst2 = jnp.concatenate([dst, n + (ar % PAD_ROWS)]).reshape(-1, GROUP)
    zeros32 = jnp.zeros((z_sl, CHUNK), jnp.float32)
    zeros1 = jnp.zeros((z_sl,), jnp.float32)

    agg2 = _make_agg(2, n_acc, n_out, e_pad, with_count=True)
    agg4 = _make_agg(4, n_acc, n_out, e_pad)

    h = x
    cnt = None
    for i in range(3):
        nch = h.shape[1] // CHUNK
        tbl = h.reshape(n * nch, CHUNK)
        if i == 0:
            agg, cnt_v = agg2(tbl, src2, dst2, zeros32, zeros1)
            cnt = cnt_v.reshape(n_out, 1)
        else:
            (agg,) = agg4(tbl, src2, dst2, zeros32)
        head = None
        if i == 2:
            head = (params['fc1_W'], params['fc1_b'],
                    params['fc2_W'], params['fc2_b'])
        h = _layer_call(agg, h, cnt, params['Wl%d' % i],
                        params['Wr%d' % i], params['bl%d' % i],
                        params['bn_g%d' % i], params['bn_b%d' % i], n,
                        head=head)
    return h
